# Initial kernel scaffold; baseline (speedup 1.0000x reference)
import jax
import jax.numpy as jnp
from jax.experimental import pallas as pl


def kernel(input_ids, tok_emb, pos_emb, latents, cw_in, cb_in, cw_out, cb_out, cln_g, cln_b, sw_in, sb_in, sw_out, sb_out, sln_g, sln_b, ln1_g, ln1_b, ln2_g, ln2_b, qkv_w, ao_w, rw, rb, nw, nb, ew1, eb1, ew2, eb2, lnf_g, lnf_b, pw, pb, hw, hb):
    raise NotImplementedError("write your pallas kernel here")


# speedup vs baseline: 1.3045x; 1.3045x over previous
import functools
import math

import jax
import jax.numpy as jnp
from jax import lax
from jax.experimental import pallas as pl
from jax.experimental.pallas import tpu as pltpu
from jax.experimental.pallas import tpu_sc as plsc

B, T, C, H, L, E, K, NL, V, PH = 2, 2048, 768, 12, 2, 8, 2, 256, 32000, 8
HD = C // H
PHD = C // PH
N = B * NL
CAP = N * K // E
EPS = 1e-5

_SC_NC, _SC_NS = 2, 16
_NW = _SC_NC * _SC_NS


def _embed_gather(tok_emb, ids):
    total = B * T
    bpw = total // _NW
    mesh = plsc.VectorSubcoreMesh(core_axis_name="c", subcore_axis_name="s")

    @functools.partial(
        pl.kernel,
        mesh=mesh,
        out_type=jax.ShapeDtypeStruct((total, C), jnp.float32),
        scratch_types=[
            pltpu.VMEM((bpw,), jnp.int32),
            pltpu.VMEM((bpw, C), jnp.float32),
            pltpu.SemaphoreType.DMA,
        ],
    )
    def k(table_hbm, idx_hbm, out_hbm, idx_v, rows_v, sem):
        wid = lax.axis_index("s") * _SC_NC + lax.axis_index("c")
        base = wid * bpw
        pltpu.sync_copy(idx_hbm.at[pl.ds(base, bpw)], idx_v)
        pltpu.async_copy(table_hbm.at[idx_v], rows_v, sem).wait()
        pltpu.sync_copy(rows_v, out_hbm.at[pl.ds(base, bpw)])

    return k(tok_emb, ids)


def _ln(x, g, b):
    m = jnp.mean(x, axis=-1, keepdims=True)
    v = jnp.mean((x - m) ** 2, axis=-1, keepdims=True)
    return (x - m) / jnp.sqrt(v + EPS) * g + b


def _softmax_rows(s):
    m = jnp.max(s, axis=-1, keepdims=True)
    p = jnp.exp(s - m)
    return p / jnp.sum(p, axis=-1, keepdims=True)


def _mha_block(q_in, kv, w_in, b_in, w_out, b_out, nheads):
    d = q_in.shape[-1]
    hd = d // nheads
    wq, wk, wv = w_in[:d], w_in[d:2 * d], w_in[2 * d:]
    q = lax.dot_general(q_in, wq, (((1,), (1,)), ((), ()))) + b_in[:, :d]
    k = lax.dot_general(kv, wk, (((1,), (1,)), ((), ()))) + b_in[:, d:2 * d]
    v = lax.dot_general(kv, wv, (((1,), (1,)), ((), ()))) + b_in[:, 2 * d:]
    scale = 1.0 / math.sqrt(hd)
    outs = []
    for h in range(nheads):
        qh = q[:, h * hd:(h + 1) * hd]
        kh = k[:, h * hd:(h + 1) * hd]
        vh = v[:, h * hd:(h + 1) * hd]
        s = lax.dot_general(qh, kh, (((1,), (1,)), ((), ()))) * scale
        outs.append(lax.dot_general(_softmax_rows(s), vh,
                                    (((1,), (0,)), ((), ()))))
    o = jnp.concatenate(outs, axis=1)
    return lax.dot_general(o, w_out, (((1,), (1,)), ((), ()))) + b_out


def _xattn_body(x_ref, pos_ref, lat_ref, win_ref, bin_ref, wout_ref,
                bout_ref, g_ref, b_ref, out_ref):
    xb = x_ref[...] + pos_ref[...]
    o = _mha_block(lat_ref[...], xb, win_ref[...], bin_ref[...],
                   wout_ref[...], bout_ref[...], PH)
    out_ref[0] = _ln(o, g_ref[...], b_ref[...])


def _sattn_body(lat_ref, win_ref, bin_ref, wout_ref, bout_ref, g_ref,
                b_ref, out_ref):
    lat = lat_ref[0]
    o = _mha_block(lat, lat, win_ref[...], bin_ref[...], wout_ref[...],
                   bout_ref[...], PH)
    out_ref[0] = _ln(o, g_ref[...], b_ref[...])


def _layer_body(lat_ref, ln1g_ref, ln1b_ref, qkvw_ref, aow_ref, ln2g_ref,
                ln2b_ref, rw_ref, rb_ref, nw_ref, nb_ref, noise_ref,
                latn_ref, h2_ref, gate_ref, pos_ref):
    qkvw = qkvw_ref[...]
    scale = 1.0 / math.sqrt(HD)
    noisy_rows = []
    for b in range(B):
        lat_b = lat_ref[b]
        h = _ln(lat_b, ln1g_ref[...], ln1b_ref[...])
        qkv = lax.dot_general(h, qkvw, (((1,), (1,)), ((), ())))
        q, k, v = qkv[:, :C], qkv[:, C:2 * C], qkv[:, 2 * C:]
        outs = []
        for hh in range(H):
            qh = q[:, hh * HD:(hh + 1) * HD]
            kh = k[:, hh * HD:(hh + 1) * HD]
            vh = v[:, hh * HD:(hh + 1) * HD]
            s = lax.dot_general(qh, kh, (((1,), (1,)), ((), ()))) * scale
            outs.append(lax.dot_general(_softmax_rows(s), vh,
                                        (((1,), (0,)), ((), ()))))
        o = jnp.concatenate(outs, axis=1)
        latn = lat_b + lax.dot_general(o, aow_ref[...],
                                       (((1,), (1,)), ((), ())))
        latn_ref[b] = latn
        h2 = _ln(latn, ln2g_ref[...], ln2b_ref[...])
        h2_ref[b * NL:(b + 1) * NL, :] = h2
        logits = lax.dot_general(h2, rw_ref[...],
                                 (((1,), (1,)), ((), ()))) + rb_ref[...]
        nlog = lax.dot_general(h2, nw_ref[...],
                               (((1,), (1,)), ((), ()))) + nb_ref[...]
        sp = jnp.maximum(nlog, 0.0) + jnp.log(1.0 + jnp.exp(-jnp.abs(nlog)))
        noisy_rows.append(logits + noise_ref[b] * sp)
    noisy = jnp.concatenate(noisy_rows, axis=0)
    m1 = jnp.max(noisy, axis=1, keepdims=True)
    mask1 = noisy == m1
    masked = jnp.where(mask1, -jnp.inf, noisy)
    m2 = jnp.max(masked, axis=1, keepdims=True)
    sel = mask1 | (masked == m2)
    ez = jnp.where(sel, jnp.exp(noisy - m1), 0.0)
    gate_ref[...] = ez / jnp.sum(ez, axis=1, keepdims=True)
    em = sel.astype(jnp.float32)
    r_io = lax.broadcasted_iota(jnp.int32, (N, N), 0)
    c_io = lax.broadcasted_iota(jnp.int32, (N, N), 1)
    tril = (c_io <= r_io).astype(jnp.float32)
    cs = lax.dot_general(tril, em, (((1,), (0,)), ((), ())))
    pos_ref[...] = jnp.where(sel, cs - 1.0, -1.0)


def _moe_body(h2_ref, gate_ref, pos_ref, latf_ref, w1_ref, b1_ref,
              w2_ref, b2_ref, out_ref):
    e = pl.program_id(0)

    @pl.when(e == 0)
    def _():
        out_ref[...] = latf_ref[...]

    colmask = lax.broadcasted_iota(jnp.int32, (N, E), 1) == e
    pe = jnp.sum(jnp.where(colmask, pos_ref[...], 0.0), axis=1,
                 keepdims=True)
    ge = jnp.sum(jnp.where(colmask, gate_ref[...], 0.0), axis=1,
                 keepdims=True)
    c_io = lax.broadcasted_iota(jnp.int32, (N, CAP), 1).astype(jnp.float32)
    pt = (pe == c_io).astype(jnp.float32)
    xg = lax.dot_general(pt, h2_ref[...], (((0,), (0,)), ((), ())))
    h1 = jnp.maximum(
        lax.dot_general(xg, w1_ref[0], (((1,), (1,)), ((), ())))
        + b1_ref[0], 0.0)
    eo = lax.dot_general(h1, w2_ref[0], (((1,), (1,)), ((), ())))
    eo = eo + b2_ref[0]
    out_ref[...] += lax.dot_general(pt * ge, eo, (((1,), (0,)), ((), ())))


def _final_body(lat_ref, g_ref, b_ref, pw_ref, pb_ref, hw_ref, hb_ref,
                out_ref):
    for b in range(B):
        xf = _ln(lat_ref[b], g_ref[...], b_ref[...])
        sc = jnp.sum(xf * pw_ref[...], axis=1, keepdims=True) + pb_ref[...]
        m = jnp.max(sc, axis=0, keepdims=True)
        p = jnp.exp(sc - m)
        p = p / jnp.sum(p, axis=0, keepdims=True)
        pooled = jnp.sum(p * xf, axis=0, keepdims=True)
        val = jnp.sum(pooled * hw_ref[...], axis=1, keepdims=True)
        out_ref[pl.ds(b, 1), :] = val + hb_ref[...]


def _row2(x):
    return x.reshape(1, -1)


def kernel(input_ids, tok_emb, pos_emb, latents, cw_in, cb_in, cw_out,
           cb_out, cln_g, cln_b, sw_in, sb_in, sw_out, sb_out, sln_g,
           sln_b, ln1_g, ln1_b, ln2_g, ln2_b, qkv_w, ao_w, rw, rb, nw,
           nb, ew1, eb1, ew2, eb2, lnf_g, lnf_b, pw, pb, hw, hb):
    ids = input_ids.reshape(-1).astype(jnp.int32)
    x_flat = _embed_gather(tok_emb, ids)

    full = lambda *shape: pl.BlockSpec(shape, lambda b: (0,) * len(shape))
    lat = pl.pallas_call(
        _xattn_body,
        grid=(B,),
        in_specs=[
            pl.BlockSpec((T, C), lambda b: (b, 0)),
            full(T, C), full(NL, C), full(3 * C, C), full(1, 3 * C),
            full(C, C), full(1, C), full(1, C), full(1, C),
        ],
        out_specs=pl.BlockSpec((1, NL, C), lambda b: (b, 0, 0)),
        out_shape=jax.ShapeDtypeStruct((B, NL, C), jnp.float32),
    )(x_flat, pos_emb, latents, cw_in, _row2(cb_in), cw_out,
      _row2(cb_out), _row2(cln_g), _row2(cln_b))

    lat = pl.pallas_call(
        _sattn_body,
        grid=(B,),
        in_specs=[
            pl.BlockSpec((1, NL, C), lambda b: (b, 0, 0)),
            full(3 * C, C), full(1, 3 * C), full(C, C), full(1, C),
            full(1, C), full(1, C),
        ],
        out_specs=pl.BlockSpec((1, NL, C), lambda b: (b, 0, 0)),
        out_shape=jax.ShapeDtypeStruct((B, NL, C), jnp.float32),
    )(lat, sw_in, _row2(sb_in), sw_out, _row2(sb_out), _row2(sln_g),
      _row2(sln_b))

    for l in range(L):
        noise = jax.random.normal(jax.random.key(100 + l), (B, NL, E),
                                  dtype=jnp.float32)
        latn, h2f, gate, posx = pl.pallas_call(
            _layer_body,
            out_shape=(
                jax.ShapeDtypeStruct((B, NL, C), jnp.float32),
                jax.ShapeDtypeStruct((N, C), jnp.float32),
                jax.ShapeDtypeStruct((N, E), jnp.float32),
                jax.ShapeDtypeStruct((N, E), jnp.float32),
            ),
        )(lat, _row2(ln1_g[l]), _row2(ln1_b[l]), qkv_w[l], ao_w[l],
          _row2(ln2_g[l]), _row2(ln2_b[l]), rw[l], _row2(rb[l]), nw[l],
          _row2(nb[l]), noise)

        latf = pl.pallas_call(
            _moe_body,
            grid=(E,),
            in_specs=[
                pl.BlockSpec((N, C), lambda e: (0, 0)),
                pl.BlockSpec((N, E), lambda e: (0, 0)),
                pl.BlockSpec((N, E), lambda e: (0, 0)),
                pl.BlockSpec((N, C), lambda e: (0, 0)),
                pl.BlockSpec((1, 4 * C, C), lambda e: (e, 0, 0)),
                pl.BlockSpec((1, 1, 4 * C), lambda e: (e, 0, 0)),
                pl.BlockSpec((1, C, 4 * C), lambda e: (e, 0, 0)),
                pl.BlockSpec((1, 1, C), lambda e: (e, 0, 0)),
            ],
            out_specs=pl.BlockSpec((N, C), lambda e: (0, 0)),
            out_shape=jax.ShapeDtypeStruct((N, C), jnp.float32),
        )(h2f, gate, posx, latn.reshape(N, C), ew1[l],
          eb1[l].reshape(E, 1, 4 * C), ew2[l], eb2[l].reshape(E, 1, C))
        lat = latf.reshape(B, NL, C)

    out = pl.pallas_call(
        _final_body,
        out_shape=jax.ShapeDtypeStruct((B, 1), jnp.float32),
    )(lat, _row2(lnf_g), _row2(lnf_b), pw, _row2(pb), hw, _row2(hb))
    return out
